# Initial kernel scaffold; baseline (speedup 1.0000x reference)
import jax
import jax.numpy as jnp
from jax.experimental import pallas as pl


def kernel(x, edge_index, W1, b1):
    raise NotImplementedError("write your pallas kernel here")


# speedup vs baseline: 3.9069x; 3.9069x over previous
import functools

import jax
import jax.numpy as jnp
from jax import lax
from jax.experimental import pallas as pl
from jax.experimental.pallas import tpu as pltpu
from jax.experimental.pallas import tpu_sc as plsc

N = 10000
D = 128
DH = D // 2
E = 320000
NC = 2
NS = 16
NT = NC * NS
CHUNK = 128
CCHUNKS = 80
SCHUNKS = 160
E_PAD = NT * CCHUNKS * CHUNK
CNT_W = 16
OSTRIPE = 624
LSTRIPE = N - (NS - 1) * OSTRIPE
SPAD = 8
TROWS = SPAD + N + 8
CLIN = 16
NBLK = 25
BLK = N // NBLK


@functools.cache
def _sc_kernels():
  mesh = plsc.VectorSubcoreMesh(core_axis_name="c", subcore_axis_name="s")
  sc_count = functools.partial(
      pl.kernel,
      out_type=jax.ShapeDtypeStruct((NC, N, CNT_W), jnp.float32),
      mesh=mesh,
      scratch_types=[
          pltpu.VMEM((CCHUNKS, CHUNK), jnp.int32),
          pltpu.VMEM((CHUNK, CNT_W), jnp.float32),
          pltpu.VMEM((LSTRIPE, CNT_W), jnp.float32),
          pltpu.VMEM_SHARED((TROWS, CNT_W), jnp.float32),
          pltpu.SemaphoreType.DMA,
      ],
  )(_sc_count_body)
  sc_scatter = functools.partial(
      pl.kernel,
      out_type=jax.ShapeDtypeStruct((NC, N, DH), jnp.float32),
      mesh=mesh,
      scratch_types=[
          pltpu.VMEM((SCHUNKS, CHUNK), jnp.int32),
          pltpu.VMEM((SCHUNKS, CHUNK), jnp.int32),
          pltpu.VMEM((CHUNK, DH), jnp.float32),
          pltpu.VMEM((CHUNK, DH), jnp.float32),
          pltpu.VMEM((CHUNK, DH), jnp.float32),
          pltpu.VMEM((CHUNK, DH), jnp.float32),
          pltpu.VMEM((CHUNK, DH), jnp.float32),
          pltpu.VMEM((LSTRIPE, DH), jnp.float32),
          pltpu.VMEM_SHARED((TROWS, DH), jnp.float32),
          pltpu.SemaphoreType.DMA,
          pltpu.SemaphoreType.DMA,
          pltpu.SemaphoreType.DMA,
          pltpu.SemaphoreType.DMA,
          pltpu.SemaphoreType.DMA,
          pltpu.SemaphoreType.DMA,
      ],
      compiler_params=pltpu.CompilerParams(use_tc_tiling_on_sc=False),
  )(_sc_scatter_body)
  return sc_count, sc_scatter


def _sc_count_body(dst_hbm, cnt_hbm, didx, ones, zb, table, sem):
  c = lax.axis_index("c")
  s = lax.axis_index("s")
  wid = c * NS + s
  pltpu.sync_copy(dst_hbm.at[wid], didx)

  @pl.loop(0, CHUNK)
  def _(i):
    ones[i, :] = jnp.full((CNT_W,), 1.0, jnp.float32)

  @pl.loop(0, LSTRIPE)
  def _(i):
    zb[i, :] = jnp.zeros((CNT_W,), jnp.float32)

  @pl.when(s < NS - 1)
  def _():
    @pl.loop(0, OSTRIPE // CLIN)
    def _(k):
      pltpu.sync_copy(zb.at[pl.ds(0, CLIN)],
                      table.at[pl.ds(SPAD + s * OSTRIPE + k * CLIN, CLIN)])

  @pl.when(s == NS - 1)
  def _():
    @pl.loop(0, LSTRIPE // CLIN)
    def _(k):
      pltpu.sync_copy(zb.at[pl.ds(0, CLIN)],
                      table.at[pl.ds(SPAD + (NS - 1) * OSTRIPE + k * CLIN,
                                     CLIN)])

  plsc.subcore_barrier()

  @pl.loop(0, CCHUNKS)
  def _(j):
    pltpu.sync_copy(ones, table.at[didx.at[j]], add=True)

  plsc.subcore_barrier()

  @pl.when(s < NS - 1)
  def _():
    @pl.loop(0, OSTRIPE // CLIN)
    def _(k):
      pltpu.sync_copy(table.at[pl.ds(SPAD + s * OSTRIPE + k * CLIN, CLIN)],
                      zb.at[pl.ds(k * CLIN, CLIN)])
    pltpu.sync_copy(zb.at[pl.ds(0, OSTRIPE)],
                    cnt_hbm.at[c, pl.ds(s * OSTRIPE, OSTRIPE)])

  @pl.when(s == NS - 1)
  def _():
    @pl.loop(0, LSTRIPE // CLIN)
    def _(k):
      pltpu.sync_copy(table.at[pl.ds(SPAD + (NS - 1) * OSTRIPE + k * CLIN,
                                     CLIN)],
                      zb.at[pl.ds(k * CLIN, CLIN)])
    pltpu.sync_copy(zb, cnt_hbm.at[c, pl.ds((NS - 1) * OSTRIPE, LSTRIPE)])


def _sc_scatter_body(y0_hbm, y1_hbm, src_hbm, dst_hbm, out_hbm,
                     sidx, didx, r0, r1, r2, r3, r4, zb, acc,
                     g0, g1, g2, g3, g4, ss):
  c = lax.axis_index("c")
  s = lax.axis_index("s")
  pltpu.sync_copy(src_hbm.at[s], sidx)
  pltpu.sync_copy(dst_hbm.at[s], didx)

  @pl.loop(0, CLIN)
  def _(i):
    for k in range(DH // 16):
      zb[i, pl.ds(k * 16, 16)] = jnp.zeros((16,), jnp.float32)

  @pl.when(s < NS - 1)
  def _():
    @pl.loop(0, OSTRIPE // CLIN)
    def _(k):
      pltpu.sync_copy(zb.at[pl.ds(0, CLIN)],
                      acc.at[pl.ds(SPAD + s * OSTRIPE + k * CLIN, CLIN)])

  @pl.when(s == NS - 1)
  def _():
    @pl.loop(0, LSTRIPE // CLIN)
    def _(k):
      pltpu.sync_copy(zb.at[pl.ds(0, CLIN)],
                      acc.at[pl.ds(SPAD + (NS - 1) * OSTRIPE + k * CLIN,
                                   CLIN)])

  plsc.subcore_barrier()

  def edge_loop(y_hbm):
    @pl.loop(0, SCHUNKS)
    def _(j):
      pltpu.async_copy(y_hbm.at[sidx.at[j]], r0, g0).wait()
      pltpu.sync_copy(r0, acc.at[didx.at[j]], add=True)

  @pl.when(c == 0)
  def _():
    edge_loop(y0_hbm)

  @pl.when(c == 1)
  def _():
    edge_loop(y1_hbm)

  plsc.subcore_barrier()

  @pl.when(s < NS - 1)
  def _():
    @pl.loop(0, OSTRIPE // CLIN)
    def _(k):
      pltpu.sync_copy(acc.at[pl.ds(SPAD + s * OSTRIPE + k * CLIN, CLIN)],
                      zb.at[pl.ds(k * CLIN, CLIN)])
    pltpu.sync_copy(zb.at[pl.ds(0, OSTRIPE)],
                    out_hbm.at[c, pl.ds(s * OSTRIPE, OSTRIPE)])

  @pl.when(s == NS - 1)
  def _():
    @pl.loop(0, LSTRIPE // CLIN)
    def _(k):
      pltpu.sync_copy(acc.at[pl.ds(SPAD + (NS - 1) * OSTRIPE + k * CLIN,
                                   CLIN)],
                      zb.at[pl.ds(k * CLIN, CLIN)])
    pltpu.sync_copy(zb, out_hbm.at[c, pl.ds((NS - 1) * OSTRIPE, LSTRIPE)])


def _tc_scale_body(x_ref, w_ref, cnt_ref, y_ref):
  xw = jnp.dot(x_ref[...], w_ref[0], preferred_element_type=jnp.float32)
  cnt = cnt_ref[0, 0, :, 0:1] + cnt_ref[1, 0, :, 0:1] + 1.0
  y_ref[0, 0] = xw * lax.rsqrt(cnt)


def _tc_combine_body(pacc_ref, y_ref, cnt_ref, b_ref, o_ref):
  cnt = cnt_ref[0, 0, :, 0:1] + cnt_ref[1, 0, :, 0:1] + 1.0
  dinv = lax.rsqrt(cnt)
  h0 = pacc_ref[0, 0] + y_ref[0, 0]
  h1 = pacc_ref[1, 0] + y_ref[1, 0]
  ssum = jnp.concatenate([h0, h1], axis=1)
  o_ref[...] = jnp.maximum(ssum * dinv + b_ref[...], 0.0)


_tc_scale = pl.pallas_call(
    _tc_scale_body,
    grid=(NBLK, NC),
    in_specs=[
        pl.BlockSpec((BLK, D), lambda i, j: (i, 0)),
        pl.BlockSpec((1, D, DH), lambda i, j: (j, 0, 0)),
        pl.BlockSpec((NC, 1, BLK, CNT_W), lambda i, j: (0, i, 0, 0)),
    ],
    out_specs=pl.BlockSpec((1, 1, BLK, DH), lambda i, j: (j, i, 0, 0)),
    out_shape=jax.ShapeDtypeStruct((NC, NBLK, BLK, DH), jnp.float32),
)

_tc_combine = pl.pallas_call(
    _tc_combine_body,
    grid=(NBLK,),
    in_specs=[
        pl.BlockSpec((NC, 1, BLK, DH), lambda i: (0, i, 0, 0)),
        pl.BlockSpec((NC, 1, BLK, DH), lambda i: (0, i, 0, 0)),
        pl.BlockSpec((NC, 1, BLK, CNT_W), lambda i: (0, i, 0, 0)),
        pl.BlockSpec((1, D), lambda i: (0, 0)),
    ],
    out_specs=pl.BlockSpec((BLK, D), lambda i: (i, 0)),
    out_shape=jax.ShapeDtypeStruct((N, D), jnp.float32),
)


def _tc_scale_full_body(x_ref, w_ref, cnt_ref, y_ref):
  xw = jnp.dot(x_ref[...], w_ref[...], preferred_element_type=jnp.float32)
  cnt = cnt_ref[0, :, 0:1] + 1.0
  y_ref[...] = xw * lax.rsqrt(cnt)


def _tc_combine_full_body(pacc_ref, y_ref, cnt_ref, b_ref, o_ref):
  cnt = cnt_ref[0, :, 0:1] + 1.0
  ssum = pacc_ref[...] + y_ref[...]
  o_ref[...] = jnp.maximum(ssum * lax.rsqrt(cnt) + b_ref[...], 0.0)


_tc_scale_full = pl.pallas_call(
    _tc_scale_full_body,
    grid=(NBLK,),
    in_specs=[
        pl.BlockSpec((BLK, D), lambda i: (i, 0)),
        pl.BlockSpec((D, D), lambda i: (0, 0)),
        pl.BlockSpec((1, BLK, CNT_W), lambda i: (i, 0, 0)),
    ],
    out_specs=pl.BlockSpec((BLK, D), lambda i: (i, 0)),
    out_shape=jax.ShapeDtypeStruct((N, D), jnp.float32),
)

_tc_combine_full = pl.pallas_call(
    _tc_combine_full_body,
    grid=(NBLK,),
    in_specs=[
        pl.BlockSpec((BLK, D), lambda i: (i, 0)),
        pl.BlockSpec((BLK, D), lambda i: (i, 0)),
        pl.BlockSpec((1, BLK, CNT_W), lambda i: (i, 0, 0)),
        pl.BlockSpec((1, D), lambda i: (0, 0)),
    ],
    out_specs=pl.BlockSpec((BLK, D), lambda i: (i, 0)),
    out_shape=jax.ShapeDtypeStruct((N, D), jnp.float32),
)


def kernel(x, edge_index, W1, b1):
  src = edge_index[0].astype(jnp.int32)
  dst = edge_index[1].astype(jnp.int32)

  cnt = jax.ops.segment_sum(jnp.ones((E,), jnp.float32), dst, num_segments=N)
  cnt16 = jnp.broadcast_to(cnt[:, None], (N, CNT_W)).reshape(
      NBLK, BLK, CNT_W)
  y = _tc_scale_full(x, W1, cnt16)
  pacc = jnp.zeros((N, D), jnp.float32).at[dst].add(y[src])
  out = _tc_combine_full(pacc, y, cnt16, b1.reshape(1, D))
  return out
